# baseline (speedup 1.0000x reference)
import functools

import jax
import jax.numpy as jnp
from jax import lax
from jax.experimental import pallas as pl
from jax.experimental.pallas import tpu as pltpu
from jax.experimental.pallas import tpu_sc as plsc

D_MODEL = 1024
NC = 2
NS = 16
NW = NC * NS
CH = 32


def _add_rows(buf, wpe_buf, rows):
    def body(r, _):
        for k in range(D_MODEL // 16):
            sl = pl.ds(k * 16, 16)
            buf[r, sl] = buf[r, sl] + wpe_buf[r, sl]
        return 0
    lax.fori_loop(0, rows, body, 0)


def _emb_body(batch, seq, ids_hbm, am_hbm, wte_hbm, wpe_hbm,
              out_hbm, mask_hbm, idx_v, buf, wpe_buf, am_v, sem):
    wid = lax.axis_index("s") * NC + lax.axis_index("c")
    pos_w = seq // NW
    pos_base = wid * pos_w

    n_tok = batch * seq
    am_per_w = n_tok // NW
    am_base = wid * am_per_w
    pltpu.sync_copy(am_hbm.at[pl.ds(am_base, am_per_w)], am_v)
    for i in range(am_per_w // 16):
        v = am_v[pl.ds(i * 16, 16)]
        am_v[pl.ds(i * 16, 16)] = (1.0 - v) * (-10000.0)
    pltpu.sync_copy(am_v, mask_hbm.at[pl.ds(am_base, am_per_w)])

    for h in range(pos_w // CH):
        pltpu.sync_copy(wpe_hbm.at[pl.ds(pos_base + h * CH, CH)], wpe_buf)
        for b in range(batch):
            row0 = b * seq + pos_base + h * CH
            pltpu.sync_copy(ids_hbm.at[pl.ds(row0, CH)], idx_v)
            pltpu.async_copy(wte_hbm.at[idx_v], buf, sem).wait()
            _add_rows(buf, wpe_buf, CH)
            pltpu.sync_copy(buf, out_hbm.at[pl.ds(row0, CH)])


@functools.lru_cache(maxsize=None)
def _build(batch, seq):
    mesh = plsc.VectorSubcoreMesh(core_axis_name="c", subcore_axis_name="s")
    n_tok = batch * seq
    return pl.kernel(
        functools.partial(_emb_body, batch, seq),
        out_type=(
            jax.ShapeDtypeStruct((n_tok, D_MODEL), jnp.float32),
            jax.ShapeDtypeStruct((n_tok,), jnp.float32),
        ),
        mesh=mesh,
        scratch_types=[
            pltpu.VMEM((CH,), jnp.int32),
            pltpu.VMEM((CH, D_MODEL), jnp.float32),
            pltpu.VMEM((CH, D_MODEL), jnp.float32),
            pltpu.VMEM((n_tok // NW,), jnp.float32),
            pltpu.SemaphoreType.DMA,
        ],
    )


def kernel(input_ids, attention_mask, wte, wpe):
    batch, seq = input_ids.shape
    n_tok = batch * seq
    ids_flat = input_ids.reshape(n_tok).astype(jnp.int32)
    am_flat = attention_mask.reshape(n_tok).astype(jnp.float32)
    hidden_flat, mask_flat = _build(batch, seq)(ids_flat, am_flat, wte, wpe)
    hidden = hidden_flat.reshape(batch, seq, D_MODEL)
    ext_mask = mask_flat.reshape(1, 1, batch, seq)
    return (hidden, ext_mask)


# speedup vs baseline: 1.2771x; 1.2771x over previous
import functools

import jax
import jax.numpy as jnp
from jax import lax
from jax.experimental import pallas as pl
from jax.experimental.pallas import tpu as pltpu
from jax.experimental.pallas import tpu_sc as plsc

D_MODEL = 1024
NC = 2
NS = 16
NW = NC * NS
CH = 16
NBUF = 3


def _emb_body(batch, seq, ids_hbm, am_hbm, wte_hbm, wpe_hbm,
              out_hbm, mask_hbm, idx_v, buf0, buf1, buf2, wpe_buf, am_v,
              gs0, gs1, gs2, os0, os1, os2, wsem):
    bufs = (buf0, buf1, buf2)
    gsem = (gs0, gs1, gs2)
    osem = (os0, os1, os2)
    wid = lax.axis_index("s") * NC + lax.axis_index("c")
    pos_w = seq // NW
    pos_base = wid * pos_w
    hmax = pos_w // CH
    n_chunks = batch * hmax

    wpe_cp = pltpu.async_copy(wpe_hbm.at[pl.ds(pos_base, pos_w)], wpe_buf, wsem)
    for b in range(batch):
        pltpu.sync_copy(ids_hbm.at[pl.ds(b * seq + pos_base, pos_w)],
                        idx_v.at[pl.ds(b * pos_w, pos_w)])

    def gather(i):
        s = i % NBUF
        b, h = i // hmax, i % hmax
        pltpu.async_copy(
            wte_hbm.at[idx_v.at[pl.ds(b * pos_w + h * CH, CH)]],
            bufs[s], gsem[s])

    gather(0)
    gather(1)

    n_tok = batch * seq
    am_per_w = n_tok // NW
    am_base = wid * am_per_w
    pltpu.sync_copy(am_hbm.at[pl.ds(am_base, am_per_w)], am_v)
    for i in range(am_per_w // 16):
        v = am_v[pl.ds(i * 16, 16)]
        am_v[pl.ds(i * 16, 16)] = (1.0 - v) * (-10000.0)
    pltpu.sync_copy(am_v, mask_hbm.at[pl.ds(am_base, am_per_w)])

    wpe_cp.wait()

    for i in range(n_chunks):
        s = i % NBUF
        b, h = i // hmax, i % hmax
        pltpu.make_async_copy(
            wte_hbm.at[idx_v.at[pl.ds(b * pos_w + h * CH, CH)]],
            bufs[s], gsem[s]).wait()

        wrow = h * CH

        def add_row(r, _, buf=bufs[s], wrow=wrow):
            for k in range(D_MODEL // 16):
                sl = pl.ds(k * 16, 16)
                plsc.addupdate(buf.at[r, sl], wpe_buf[wrow + r, sl])
            return 0
        lax.fori_loop(0, CH, add_row, 0)

        row0 = b * seq + pos_base + h * CH
        out_cp = pltpu.async_copy(bufs[s], out_hbm.at[pl.ds(row0, CH)], osem[s])
        if i + 2 < n_chunks:
            s2 = (i + 2) % NBUF
            if i - 1 >= 0:
                b2, h2 = (i - 1) // hmax, (i - 1) % hmax
                row2 = b2 * seq + pos_base + h2 * CH
                pltpu.make_async_copy(
                    bufs[s2], out_hbm.at[pl.ds(row2, CH)], osem[s2]).wait()
            gather(i + 2)
        if i >= n_chunks - 2:
            out_cp.wait()


@functools.lru_cache(maxsize=None)
def _build(batch, seq):
    mesh = plsc.VectorSubcoreMesh(core_axis_name="c", subcore_axis_name="s")
    n_tok = batch * seq
    pos_w = seq // NW
    return pl.kernel(
        functools.partial(_emb_body, batch, seq),
        out_type=(
            jax.ShapeDtypeStruct((n_tok, D_MODEL), jnp.float32),
            jax.ShapeDtypeStruct((n_tok,), jnp.float32),
        ),
        mesh=mesh,
        scratch_types=[
            pltpu.VMEM((batch * pos_w,), jnp.int32),
            pltpu.VMEM((CH, D_MODEL), jnp.float32),
            pltpu.VMEM((CH, D_MODEL), jnp.float32),
            pltpu.VMEM((CH, D_MODEL), jnp.float32),
            pltpu.VMEM((pos_w, D_MODEL), jnp.float32),
            pltpu.VMEM((n_tok // NW,), jnp.float32),
            pltpu.SemaphoreType.DMA,
            pltpu.SemaphoreType.DMA,
            pltpu.SemaphoreType.DMA,
            pltpu.SemaphoreType.DMA,
            pltpu.SemaphoreType.DMA,
            pltpu.SemaphoreType.DMA,
            pltpu.SemaphoreType.DMA,
        ],
    )


def kernel(input_ids, attention_mask, wte, wpe):
    batch, seq = input_ids.shape
    n_tok = batch * seq
    ids_flat = input_ids.reshape(n_tok).astype(jnp.int32)
    am_flat = attention_mask.reshape(n_tok).astype(jnp.float32)
    hidden_flat, mask_flat = _build(batch, seq)(ids_flat, am_flat, wte, wpe)
    hidden = hidden_flat.reshape(batch, seq, D_MODEL)
    ext_mask = mask_flat.reshape(1, 1, batch, seq)
    return (hidden, ext_mask)


# speedup vs baseline: 1.4706x; 1.1515x over previous
import functools

import jax
import jax.numpy as jnp
from jax import lax
from jax.experimental import pallas as pl
from jax.experimental.pallas import tpu as pltpu
from jax.experimental.pallas import tpu_sc as plsc

D_MODEL = 1024
NC = 2
NS = 16
NW = NC * NS
CH = 16
NBUF = 5


def _emb_body(batch, seq, ids_hbm, am_hbm, wte_hbm, wpe_hbm,
              out_hbm, mask_hbm, idx_v, b0, b1, b2, b3, b4, wq0, wq1, am_v,
              g0, g1, g2, g3, g4, o0, o1, o2, o3, o4, w0, w1):
    bufs = (b0, b1, b2, b3, b4)
    gsem = (g0, g1, g2, g3, g4)
    osem = (o0, o1, o2, o3, o4)
    wq = (wq0, wq1)
    wsem = (w0, w1)
    wid = lax.axis_index("s") * NC + lax.axis_index("c")
    pos_w = seq // NW
    pos_base = wid * pos_w
    nq = pos_w // CH
    n_chunks = batch * nq

    def load_wq(q):
        pltpu.async_copy(wpe_hbm.at[pl.ds(pos_base + q * CH, CH)],
                         wq[q % 2], wsem[q % 2])

    load_wq(0)
    load_wq(1)
    for b in range(batch):
        pltpu.sync_copy(ids_hbm.at[b, pl.ds(pos_base, pos_w)],
                        idx_v.at[pl.ds(b * pos_w, pos_w)])

    def gather(i):
        q, b = i // batch, i % batch
        s = i % NBUF
        pltpu.async_copy(
            wte_hbm.at[idx_v.at[pl.ds(b * pos_w + q * CH, CH)]],
            bufs[s], gsem[s])

    for i in range(min(NBUF - 1, n_chunks)):
        gather(i)

    wpb = seq // 256
    mb = wid // wpb
    mcol = (wid % wpb) * 256
    pltpu.sync_copy(am_hbm.at[mb, pl.ds(mcol, 256)], am_v)
    for i in range(256 // 16):
        v = am_v[pl.ds(i * 16, 16)]
        am_v[pl.ds(i * 16, 16)] = (1.0 - v) * (-10000.0)
    pltpu.sync_copy(am_v, mask_hbm.at[0, 0, mb, pl.ds(mcol, 256)])

    for i in range(n_chunks):
        q, b = i // batch, i % batch
        s = i % NBUF
        if b == 0:
            pltpu.make_async_copy(
                wpe_hbm.at[pl.ds(pos_base + q * CH, CH)],
                wq[q % 2], wsem[q % 2]).wait()
        pltpu.make_async_copy(
            wte_hbm.at[idx_v.at[pl.ds(b * pos_w + q * CH, CH)]],
            bufs[s], gsem[s]).wait()

        def add_row(r, _, buf=bufs[s], w=wq[q % 2]):
            for k in range(D_MODEL // 16):
                sl = pl.ds(k * 16, 16)
                plsc.addupdate(buf.at[r, sl], w[r, sl])
            return 0
        lax.fori_loop(0, CH, add_row, 0)

        row0 = pos_base + q * CH
        out_cp = pltpu.async_copy(bufs[s], out_hbm.at[b, pl.ds(row0, CH)],
                                  osem[s])
        if b == batch - 1 and q + 2 < nq:
            load_wq(q + 2)
        if i + NBUF - 1 < n_chunks:
            j = i + NBUF - 1
            jp = j - NBUF
            if jp >= 0:
                qp, bp = jp // batch, jp % batch
                pltpu.make_async_copy(
                    bufs[jp % NBUF],
                    out_hbm.at[bp, pl.ds(pos_base + qp * CH, CH)],
                    osem[jp % NBUF]).wait()
            gather(j)
        if i >= n_chunks - NBUF:
            out_cp.wait()


@functools.lru_cache(maxsize=None)
def _build(batch, seq):
    mesh = plsc.VectorSubcoreMesh(core_axis_name="c", subcore_axis_name="s")
    pos_w = seq // NW
    return pl.kernel(
        functools.partial(_emb_body, batch, seq),
        out_type=(
            jax.ShapeDtypeStruct((batch, seq, D_MODEL), jnp.float32),
            jax.ShapeDtypeStruct((1, 1, batch, seq), jnp.float32),
        ),
        mesh=mesh,
        scratch_types=[
            pltpu.VMEM((batch * pos_w,), jnp.int32),
            pltpu.VMEM((CH, D_MODEL), jnp.float32),
            pltpu.VMEM((CH, D_MODEL), jnp.float32),
            pltpu.VMEM((CH, D_MODEL), jnp.float32),
            pltpu.VMEM((CH, D_MODEL), jnp.float32),
            pltpu.VMEM((CH, D_MODEL), jnp.float32),
            pltpu.VMEM((CH, D_MODEL), jnp.float32),
            pltpu.VMEM((CH, D_MODEL), jnp.float32),
            pltpu.VMEM((256,), jnp.float32),
        ] + [pltpu.SemaphoreType.DMA] * 12,
    )


def kernel(input_ids, attention_mask, wte, wpe):
    batch, seq = input_ids.shape
    ids = input_ids if input_ids.dtype == jnp.int32 else input_ids.astype(jnp.int32)
    am = (attention_mask if attention_mask.dtype == jnp.float32
          else attention_mask.astype(jnp.float32))
    hidden, ext_mask = _build(batch, seq)(ids, am, wte, wpe)
    return (hidden, ext_mask)
